# baseline (speedup 1.0000x reference)
import jax
import jax.numpy as jnp
from jax import lax
from jax.experimental import pallas as pl
from jax.experimental.pallas import tpu as pltpu
from jax.experimental.pallas import tpu_sc as plsc

B, T, N, D = 16, 4, 1024, 256
IN_N = 512
TC_B = 8
SC_B = B - TC_B
NC, NS = 2, 16
NW = NC * NS
CHUNKS = NW // T
OUT_ROWS = N // CHUNKS
IN_ROWS = OUT_ROWS // 2
BUF_ROWS = IN_ROWS + 16
LANES = 16
VPR = D // LANES
SUB = 4
SUB_J = IN_ROWS // SUB


def _sc_body(pe_hbm, out_hbm, in_v, out_v, sem):
    wid = lax.axis_index("s") * NC + lax.axis_index("c")
    tt = wid // CHUNKS
    c = wid % CHUNKS
    k0 = c * IN_ROWS
    start = pl.multiple_of(jnp.clip(k0 - 8, 0, IN_N - BUF_ROWS), 8)
    pltpu.sync_copy(pe_hbm.at[0, tt, pl.ds(start, BUF_ROWS)], in_v)

    w_lo = jnp.full((LANES,), 0.25, jnp.float32)
    w_hi = jnp.full((LANES,), 0.75, jnp.float32)

    def j_body(j, carry):
        row_a = jnp.maximum(k0 + j - 1, 0) - start
        row_b = k0 + j - start
        row_c = jnp.minimum(k0 + j + 1, IN_N - 1) - start
        for v in range(VPR):
            sl = pl.ds(v * LANES, LANES)
            a = in_v[row_a, sl]
            bb = in_v[row_b, sl]
            cc = in_v[row_c, sl]
            out_v[2 * j, sl] = w_lo * a + w_hi * bb
            out_v[2 * j + 1, sl] = w_hi * bb + w_lo * cc
        return carry

    i0 = c * OUT_ROWS
    copies = []
    for s in range(SUB):
        lax.fori_loop(s * SUB_J, (s + 1) * SUB_J, j_body, 0)
        src = out_v.at[pl.ds(s * 2 * SUB_J, 2 * SUB_J)]
        dst_row = pl.multiple_of(i0 + s * 2 * SUB_J, 8)
        for b in range(SC_B):
            copies.append(
                pltpu.async_copy(
                    src, out_hbm.at[b, tt, pl.ds(dst_row, 2 * SUB_J)], sem
                )
            )
    for cp in copies:
        cp.wait()


def _sc_call(pos_embedding):
    mesh = plsc.VectorSubcoreMesh(core_axis_name="c", subcore_axis_name="s")
    f = pl.kernel(
        _sc_body,
        out_type=jax.ShapeDtypeStruct((SC_B, T, N, D), jnp.float32),
        mesh=mesh,
        scratch_types=[
            pltpu.VMEM((BUF_ROWS, D), jnp.float32),
            pltpu.VMEM((OUT_ROWS, D), jnp.float32),
            pltpu.SemaphoreType.DMA,
        ],
    )
    return f(pos_embedding)


def _tc_body(pe_ref, out_ref, scratch):
    @pl.when(pl.program_id(1) == 0)
    def _():
        pe = pe_ref[0, 0]
        prev = jnp.concatenate([pe[:1], pe[:-1]], axis=0)
        nxt = jnp.concatenate([pe[1:], pe[-1:]], axis=0)
        even = 0.25 * prev + 0.75 * pe
        odd = 0.75 * pe + 0.25 * nxt
        scratch[...] = jnp.stack([even, odd], axis=1).reshape(N, D)

    out_ref[0, 0] = scratch[...]


def _tc_call(pos_embedding):
    return pl.pallas_call(
        _tc_body,
        grid=(T, TC_B),
        in_specs=[
            pl.BlockSpec((1, 1, IN_N, D), lambda t, b: (0, t, 0, 0)),
        ],
        out_specs=pl.BlockSpec((1, 1, N, D), lambda t, b: (b, t, 0, 0)),
        out_shape=jax.ShapeDtypeStruct((TC_B, T, N, D), jnp.float32),
        scratch_shapes=[pltpu.VMEM((N, D), jnp.float32)],
    )(pos_embedding)


def kernel(x, pos_embedding):
    tc_out = _tc_call(pos_embedding)
    sc_out = _sc_call(pos_embedding)
    return jnp.concatenate([tc_out, sc_out], axis=0)


# speedup vs baseline: 1.9611x; 1.9611x over previous
import jax
import jax.numpy as jnp
from jax import lax
from jax.experimental import pallas as pl
from jax.experimental.pallas import tpu as pltpu
from jax.experimental.pallas import tpu_sc as plsc

B, T, N, D = 16, 4, 1024, 256
IN_N = 512
NC, NS = 2, 16
NW = NC * NS
CHUNKS = NW // T
OUT_ROWS = N // CHUNKS
IN_ROWS = OUT_ROWS // 2
BUF_ROWS = IN_ROWS + 16
LANES = 16
VPR = D // LANES
SUB = 4
SUB_J = IN_ROWS // SUB


def _body(pe_hbm, out_hbm, in_v, out_v, sem):
    wid = lax.axis_index("s") * NC + lax.axis_index("c")
    tt = wid // CHUNKS
    c = wid % CHUNKS
    k0 = c * IN_ROWS
    start = pl.multiple_of(jnp.clip(k0 - 8, 0, IN_N - BUF_ROWS), 8)
    pltpu.sync_copy(pe_hbm.at[0, tt, pl.ds(start, BUF_ROWS)], in_v)

    w_lo = jnp.full((LANES,), 0.25, jnp.float32)
    w_hi = jnp.full((LANES,), 0.75, jnp.float32)

    def j_body(j, carry):
        row_a = jnp.maximum(k0 + j - 1, 0) - start
        row_b = k0 + j - start
        row_c = jnp.minimum(k0 + j + 1, IN_N - 1) - start
        for v in range(VPR):
            sl = pl.ds(v * LANES, LANES)
            a = in_v[row_a, sl]
            bb = in_v[row_b, sl]
            cc = in_v[row_c, sl]
            out_v[2 * j, sl] = w_lo * a + w_hi * bb
            out_v[2 * j + 1, sl] = w_hi * bb + w_lo * cc
        return carry

    i0 = c * OUT_ROWS
    copies = []
    for s in range(SUB):
        lax.fori_loop(s * SUB_J, (s + 1) * SUB_J, j_body, 0)
        src = out_v.at[pl.ds(s * 2 * SUB_J, 2 * SUB_J)]
        dst_row = pl.multiple_of(i0 + s * 2 * SUB_J, 8)
        for b in range(B):
            copies.append(
                pltpu.async_copy(
                    src, out_hbm.at[b, tt, pl.ds(dst_row, 2 * SUB_J)], sem
                )
            )
    for cp in copies:
        cp.wait()


def kernel(x, pos_embedding):
    mesh = plsc.VectorSubcoreMesh(core_axis_name="c", subcore_axis_name="s")
    f = pl.kernel(
        _body,
        out_type=jax.ShapeDtypeStruct((B, T, N, D), jnp.float32),
        mesh=mesh,
        scratch_types=[
            pltpu.VMEM((BUF_ROWS, D), jnp.float32),
            pltpu.VMEM((OUT_ROWS, D), jnp.float32),
            pltpu.SemaphoreType.DMA,
        ],
    )
    return f(pos_embedding)
